# baseline (speedup 1.0000x reference)
import functools

import jax
import jax.numpy as jnp
from jax import lax
from jax.experimental import pallas as pl
from jax.experimental.pallas import tpu as pltpu
from jax.experimental.pallas import tpu_sc as plsc

N = 10000
NPAD = 10240
E = 320000
D = 128
NC = 2
NS = 16
NW = NC * NS
EPW = E // NW
RCHUNK = 125
CHUNK = 128
NCHUNK = EPW // RCHUNK
TRASH = NPAD - 8
RING = 4
RPT = NPAD // NS
ZROWS = 64
BM = 1280
NB = NPAD // BM
VL = 16


def _mesh():
    return plsc.VectorSubcoreMesh(core_axis_name="c", subcore_axis_name="s")


@functools.partial(
    pl.kernel,
    out_type=jax.ShapeDtypeStruct((NC * NPAD, D), jnp.float32),
    mesh=_mesh(),
    scratch_types=[
        pltpu.VMEM((NCHUNK, CHUNK), jnp.int32),
        pltpu.VMEM((CHUNK, D), jnp.float32),
        pltpu.VMEM((ZROWS, D), jnp.float32),
        pltpu.VMEM_SHARED((NPAD, D), jnp.float32),
        pltpu.SemaphoreType.DMA,
    ],
)
def _deg_kernel(col_hbm, degp_hbm, cidx_v, ones_v, zbuf_v, acc_sh, ssem):
    c = lax.axis_index("c")
    s = lax.axis_index("s")
    wid = s * NC + c

    pltpu.sync_copy(col_hbm.at[wid], cidx_v)

    zeros16 = jnp.zeros((VL,), jnp.float32)
    ones16 = jnp.ones((VL,), jnp.float32)

    def fill_z(i, carry):
        for j in range(D // VL):
            zbuf_v[i, pl.ds(j * VL, VL)] = zeros16
        return carry

    lax.fori_loop(0, ZROWS, fill_z, 0)

    def fill_o(i, carry):
        for j in range(D // VL):
            ones_v[i, pl.ds(j * VL, VL)] = ones16
        return carry

    lax.fori_loop(0, CHUNK, fill_o, 0)

    r0 = s * RPT
    for i in range(RPT // ZROWS):
        pltpu.sync_copy(zbuf_v, acc_sh.at[pl.ds(r0 + i * ZROWS, ZROWS)])
    plsc.subcore_barrier()

    def chunk(j, carry):
        pltpu.async_copy(ones_v, acc_sh.at[cidx_v.at[j]], ssem, add=True)
        return carry

    lax.fori_loop(0, NCHUNK, chunk, 0)

    def drain(j, carry):
        pltpu.make_async_copy(ones_v, acc_sh.at[cidx_v.at[j]], ssem).wait()
        return carry

    lax.fori_loop(0, NCHUNK, drain, 0)
    plsc.subcore_barrier()

    pltpu.sync_copy(acc_sh.at[pl.ds(r0, RPT)],
                    degp_hbm.at[pl.ds(c * NPAD + r0, RPT)])


@functools.partial(
    pl.kernel,
    out_type=jax.ShapeDtypeStruct((NC * NPAD, D), jnp.float32),
    mesh=_mesh(),
    scratch_types=[
        pltpu.VMEM((NCHUNK, CHUNK), jnp.int32),
        pltpu.VMEM((RING, CHUNK), jnp.int32),
        pltpu.VMEM((RING, CHUNK), jnp.int32),
        pltpu.VMEM((CHUNK, D), jnp.float32),
        pltpu.VMEM((CHUNK, D), jnp.float32),
        pltpu.VMEM_SHARED((NPAD, D), jnp.float32),
        pltpu.SemaphoreType.DMA,
        pltpu.SemaphoreType.DMA,
        pltpu.SemaphoreType.DMA,
        pltpu.SemaphoreType.DMA,
    ],
)
def _agg_kernel(y_hbm, pidx_hbm, aggp_hbm,
                pidx_v, rbuf_v, cbuf_v, buf0_v, buf1_v, acc_sh,
                gsem0, gsem1, ssem0, ssem1):
    c = lax.axis_index("c")
    s = lax.axis_index("s")
    wid = s * NC + c

    bufs = (buf0_v, buf1_v)
    gsems = (gsem0, gsem1)
    ssems = (ssem0, ssem1)

    pltpu.sync_copy(pidx_hbm.at[wid], pidx_v)

    def unpack(k, t):
        for i in range(CHUNK // VL):
            v = pidx_v[k, pl.ds(i * VL, VL)]
            rbuf_v[t, pl.ds(i * VL, VL)] = jnp.bitwise_and(v, 0xFFFF)
            cbuf_v[t, pl.ds(i * VL, VL)] = lax.shift_right_logical(v, 16)
        return None

    def _gather(j, bslot, rslot):
        pltpu.async_copy(y_hbm.at[rbuf_v.at[rslot]], bufs[bslot],
                         gsems[bslot])

    def _wait_gather(bslot, rslot):
        pltpu.make_async_copy(y_hbm.at[rbuf_v.at[rslot]], bufs[bslot],
                              gsems[bslot]).wait()

    def _scatter(j, bslot, cslot):
        pltpu.async_copy(bufs[bslot], acc_sh.at[cbuf_v.at[cslot]],
                         ssems[bslot], add=True)

    def _wait_scatter(bslot, cslot):
        pltpu.make_async_copy(bufs[bslot], acc_sh.at[cbuf_v.at[cslot]],
                              ssems[bslot]).wait()

    unpack(0, 0)
    _gather(0, 0, 0)

    zeros16 = jnp.zeros((VL,), jnp.float32)

    def fill_z(i, carry):
        for j in range(D // VL):
            buf1_v[i, pl.ds(j * VL, VL)] = zeros16
        return carry

    lax.fori_loop(0, ZROWS, fill_z, 0)

    r0 = s * RPT
    for i in range(RPT // ZROWS):
        pltpu.sync_copy(buf1_v.at[pl.ds(0, ZROWS)],
                        acc_sh.at[pl.ds(r0 + i * ZROWS, ZROWS)])
    unpack(1, 1)
    _gather(1, 1, 1)
    plsc.subcore_barrier()

    unpack(2, 2)
    _wait_gather(0, 0)
    _scatter(0, 0, 0)

    unpack(3, 3)
    _wait_scatter(0, 0)
    _gather(2, 0, 2)
    _wait_gather(1, 1)
    _scatter(1, 1, 1)

    unpack(4, 0)
    _wait_scatter(1, 1)
    _gather(3, 1, 3)
    _wait_gather(0, 2)
    _scatter(2, 0, 2)

    unpack(5, 1)
    _wait_scatter(0, 2)
    _gather(4, 0, 0)
    _wait_gather(1, 3)
    _scatter(3, 1, 3)

    def rounds(r, carry):
        for b in range(4):
            j = 4 * r + b
            bslot = b % 2
            oslot = (b + 1) % 2
            unpack(j + 2, (b + 2) % 4)
            _wait_scatter(oslot, (b + 3) % 4)
            _gather(j + 1, oslot, (b + 1) % 4)
            _wait_gather(bslot, b)
            _scatter(j, bslot, b)
        return carry

    lax.fori_loop(1, NCHUNK // 4 - 1, rounds, 0)

    unpack(NCHUNK - 2, 2)
    _wait_scatter(1, 3)
    _gather(NCHUNK - 3, 1, 1)
    _wait_gather(0, 0)
    _scatter(NCHUNK - 4, 0, 0)

    unpack(NCHUNK - 1, 3)
    _wait_scatter(0, 0)
    _gather(NCHUNK - 2, 0, 2)
    _wait_gather(1, 1)
    _scatter(NCHUNK - 3, 1, 1)

    _wait_scatter(1, 1)
    _gather(NCHUNK - 1, 1, 3)
    _wait_gather(0, 2)
    _scatter(NCHUNK - 2, 0, 2)

    _wait_scatter(0, 2)
    _wait_gather(1, 3)
    _scatter(NCHUNK - 1, 1, 3)
    _wait_scatter(1, 3)
    plsc.subcore_barrier()

    pltpu.sync_copy(acc_sh.at[pl.ds(r0, RPT)],
                    aggp_hbm.at[pl.ds(c * NPAD + r0, RPT)])


def _dis(d0_ref, d1_ref):
    return lax.rsqrt(1.0 + d0_ref[:, 0:1] + d1_ref[:, 0:1])


def _mm_scale_body(x_ref, w_ref, d0_ref, d1_ref, y_ref):
    dis = _dis(d0_ref, d1_ref)
    y_ref[...] = jnp.dot(x_ref[...], w_ref[...],
                         preferred_element_type=jnp.float32) * dis


def _tc_layer1(x, W1, degp):
    return pl.pallas_call(
        _mm_scale_body,
        grid=(NB,),
        in_specs=[
            pl.BlockSpec((BM, D), lambda i: (i, 0)),
            pl.BlockSpec((D, D), lambda i: (0, 0)),
            pl.BlockSpec((BM, D), lambda i: (i, 0)),
            pl.BlockSpec((BM, D), lambda i: (NB + i, 0)),
        ],
        out_specs=pl.BlockSpec((BM, D), lambda i: (i, 0)),
        out_shape=jax.ShapeDtypeStruct((NPAD, D), jnp.float32),
    )(x, W1, degp, degp)


def _combine_mm_body(a0_ref, a1_ref, y1_ref, d0_ref, d1_ref, w_ref, b_ref,
                     y2_ref):
    dis = _dis(d0_ref, d1_ref)
    h = jnp.maximum(
        dis * (a0_ref[...] + a1_ref[...] + y1_ref[...]) + b_ref[...], 0.0)
    y2_ref[...] = jnp.dot(h, w_ref[...],
                          preferred_element_type=jnp.float32) * dis


def _tc_layer2(aggp, y1, degp, W2, b1):
    return pl.pallas_call(
        _combine_mm_body,
        grid=(NB,),
        in_specs=[
            pl.BlockSpec((BM, D), lambda i: (i, 0)),
            pl.BlockSpec((BM, D), lambda i: (NB + i, 0)),
            pl.BlockSpec((BM, D), lambda i: (i, 0)),
            pl.BlockSpec((BM, D), lambda i: (i, 0)),
            pl.BlockSpec((BM, D), lambda i: (NB + i, 0)),
            pl.BlockSpec((D, D), lambda i: (0, 0)),
            pl.BlockSpec((1, D), lambda i: (0, 0)),
        ],
        out_specs=pl.BlockSpec((BM, D), lambda i: (i, 0)),
        out_shape=jax.ShapeDtypeStruct((NPAD, D), jnp.float32),
    )(aggp, aggp, y1, degp, degp, W2, b1)


def _final_body(a0_ref, a1_ref, y2_ref, d0_ref, d1_ref, b_ref, o_ref):
    dis = _dis(d0_ref, d1_ref)
    o_ref[...] = jnp.maximum(
        dis * (a0_ref[...] + a1_ref[...] + y2_ref[...]) + b_ref[...], 0.0)


def _tc_final(aggp, y2, degp, b2):
    return pl.pallas_call(
        _final_body,
        grid=(NB,),
        in_specs=[
            pl.BlockSpec((BM, D), lambda i: (i, 0)),
            pl.BlockSpec((BM, D), lambda i: (NB + i, 0)),
            pl.BlockSpec((BM, D), lambda i: (i, 0)),
            pl.BlockSpec((BM, D), lambda i: (i, 0)),
            pl.BlockSpec((BM, D), lambda i: (NB + i, 0)),
            pl.BlockSpec((1, D), lambda i: (0, 0)),
        ],
        out_specs=pl.BlockSpec((BM, D), lambda i: (i, 0)),
        out_shape=jax.ShapeDtypeStruct((NPAD, D), jnp.float32),
    )(aggp, aggp, y2, degp, degp, b2)


def kernel(x, edge_index, W1, b1, W2, b2):
    ei = edge_index.astype(jnp.int32)
    row = ei[0].reshape(NW, NCHUNK, RCHUNK)
    col = ei[1].reshape(NW, NCHUNK, RCHUNK)
    row = jnp.pad(row, ((0, 0), (0, 0), (0, CHUNK - RCHUNK)))
    col = jnp.pad(col, ((0, 0), (0, 0), (0, CHUNK - RCHUNK)),
                  constant_values=TRASH)
    pidx = jnp.bitwise_or(row, col << 16)
    xp = jnp.pad(x, ((0, NPAD - N), (0, 0)))
    degp = _deg_kernel(col)
    y1 = _tc_layer1(xp, W1, degp)
    a1 = _agg_kernel(y1, pidx)
    y2 = _tc_layer2(a1, y1, degp, W2, b1.reshape(1, D))
    a2 = _agg_kernel(y2, pidx)
    return _tc_final(a2, y2, degp, b2.reshape(1, D))[:N]
